# Initial kernel scaffold; baseline (speedup 1.0000x reference)
import jax
import jax.numpy as jnp
from jax.experimental import pallas as pl


def kernel(node_feat_frags, edge_index_frags, edge_features, current_wid, next_motif_wid, focal_sca, focal_vec, node_batch_frags, node_emb_W, node_emb_b, gat0_W, gat0_edgeW, gat0_asrc, gat0_adst, gat1_W, gat1_edgeW, gat1_asrc, gat1_adst, bn0_g, bn0_b, bn1_g, bn1_b, frag_table, focal_Wv, focal_Ws, focal_bs, pred_W, pred_b):
    raise NotImplementedError("write your pallas kernel here")


# speedup vs baseline: 6.9864x; 6.9864x over previous
import functools

import jax
import jax.numpy as jnp
from jax import lax
from jax.experimental import pallas as pl
from jax.experimental.pallas import tpu as pltpu
from jax.experimental.pallas import tpu_sc as plsc

N = 10000
E = 320000
EMB = 128
NACC = 10240
NCORES = 2
NSUB = 16
CHUNK = 128
EPW = 20096
NCHUNKS = EPW // CHUNK
E_PAD = EPW * NSUB
ROWS_PER_SUB = NACC // NSUB


def _make_sc_layer(Pg):
    P = Pg + 16
    mesh = plsc.VectorSubcoreMesh(
        core_axis_name="c", subcore_axis_name="s",
        num_cores=NCORES, num_subcores=NSUB)

    @functools.partial(
        pl.kernel,
        out_type=jax.ShapeDtypeStruct((NCORES, NACC, P), jnp.float32),
        mesh=mesh,
        scratch_types=[
            pltpu.VMEM((2 * N,), jnp.float32),
            pltpu.VMEM((CHUNK,), jnp.int32),
            pltpu.VMEM((CHUNK,), jnp.int32),
            pltpu.VMEM((CHUNK,), jnp.float32),
            pltpu.VMEM((CHUNK, 16), jnp.float32),
            pltpu.VMEM((CHUNK, Pg), jnp.float32),
            pltpu.VMEM((CHUNK, P), jnp.float32),
            pltpu.VMEM_SHARED((NACC, P), jnp.float32),
            pltpu.SemaphoreType.DMA,
        ],
        compiler_params=pltpu.CompilerParams(
            needs_layout_passes=False, use_tc_tiling_on_sc=False),
    )
    def sc_layer(tab_h, s2_h, src_h, dst_h, eal_h, ea16_h, out_h,
                 s2_v, src_v, dst_v, eal_v, ea16_v, gath_v, payload,
                 acc, sem):
        c = lax.axis_index("c")
        s = lax.axis_index("s")
        zero16 = jnp.zeros((16,), jnp.float32)

        def _zrow(r, _):
            for cc in range(P // 16):
                payload[r, pl.ds(cc * 16, 16)] = zero16
            return 0
        lax.fori_loop(0, CHUNK, _zrow, 0)
        r0 = s * ROWS_PER_SUB
        for b in range(ROWS_PER_SUB // CHUNK):
            pltpu.sync_copy(payload, acc.at[pl.ds(r0 + b * CHUNK, CHUNK)])
        pltpu.sync_copy(s2_h, s2_v)
        plsc.subcore_barrier()

        base = s * EPW

        def _chunk(g, _):
            eb = base + g * CHUNK
            pltpu.sync_copy(src_h.at[pl.ds(eb, CHUNK)], src_v)
            pltpu.sync_copy(dst_h.at[pl.ds(eb, CHUNK)], dst_v)
            pltpu.sync_copy(eal_h.at[pl.ds(eb, CHUNK)], eal_v)
            pltpu.sync_copy(ea16_h.at[pl.ds(eb, CHUNK)], ea16_v)
            pltpu.async_copy(tab_h.at[c].at[src_v], gath_v, sem).wait()

            def _grp(i, _):
                sidx = src_v[pl.ds(i * 16, 16)]
                didx = dst_v[pl.ds(i * 16, 16)]
                s1 = plsc.load_gather(s2_v, [sidx * 2])
                s2x = plsc.load_gather(s2_v, [didx * 2 + 1])
                a = s1 + s2x + eal_v[pl.ds(i * 16, 16)]
                a = jnp.maximum(a, 0.2 * a)
                exg = jnp.exp(a)
                for j in range(16):
                    exv = exg[j]
                    r = i * 16 + j
                    for cc in range(Pg // 16):
                        payload[r, pl.ds(cc * 16, 16)] = (
                            gath_v[r, pl.ds(cc * 16, 16)] * exv)
                    payload[r, pl.ds(Pg, 16)] = ea16_v[r, :] * exv
                return 0
            lax.fori_loop(0, CHUNK // 16, _grp, 0)

            pltpu.sync_copy(payload, acc.at[dst_v], add=True)
            return 0

        lax.fori_loop(0, NCHUNKS, _chunk, 0)

        plsc.subcore_barrier()
        for b in range(ROWS_PER_SUB // CHUNK):
            rr = r0 + b * CHUNK
            pltpu.sync_copy(acc.at[pl.ds(rr, CHUNK)], payload)
            pltpu.sync_copy(payload, out_h.at[c].at[pl.ds(rr, CHUNK)])

    return sc_layer


_sc_layer0 = _make_sc_layer(32)
_sc_layer1 = _make_sc_layer(64)


def _pre_body(x_ref, eac_ref, We_ref, be_ref, W0_ref, a0s_ref, a0d_ref,
              E0_ref, E1_ref, a1s_ref, s01_ref, eal01_ref):
    A0 = jnp.stack([a0s_ref[:], a0d_ref[:]], axis=1)
    Q = We_ref[:] @ (W0_ref[:] @ A0)
    cb = be_ref[:] @ (W0_ref[:] @ A0)
    s01_ref[:] = x_ref[:] @ Q + cb[None, :]
    v0 = E0_ref[:] @ a0s_ref[:]
    v1 = E1_ref[:] @ a1s_ref[:]
    V = jnp.stack([v0, v1], axis=0)
    eal = V @ eac_ref[:]
    col = lax.broadcasted_iota(jnp.int32, eal.shape, 1)
    eal01_ref[:] = jnp.where(col < E, eal, -1e30)


def _tc_pre(x, ea_cols, We, be, W0, a0s, a0d, E0, E1, a1s):
    return pl.pallas_call(
        _pre_body,
        out_shape=[jax.ShapeDtypeStruct((N, 2), jnp.float32),
                   jax.ShapeDtypeStruct((2, E_PAD), jnp.float32)],
    )(x, ea_cols, We, be, W0, a0s, a0d, E0, E1, a1s)


def _mid_body(acc_ref, We_ref, be_ref, W0_ref, E0_ref, g_ref, b_ref,
              W1_ref, a1s_ref, a1d_ref, h1s_ref, s11_ref):
    A = jnp.concatenate(
        [acc_ref[0, :N, 0:32], acc_ref[1, :N, :]], axis=1)
    M0 = We_ref[:] @ W0_ref[:]
    Mc = jnp.concatenate([
        M0,
        jnp.zeros((19, EMB), jnp.float32),
        E0_ref[:],
        (be_ref[:] @ W0_ref[:])[None, :],
        jnp.zeros((11, EMB), jnp.float32),
    ], axis=0)
    num = A @ Mc
    denom = A[:, 68:69]
    h = num / (denom + 1e-16)
    mu = jnp.mean(h, axis=0, keepdims=True)
    var = jnp.mean((h - mu) * (h - mu), axis=0, keepdims=True)
    h = (h - mu) / jnp.sqrt(var + 1e-5) * g_ref[:][None, :] + b_ref[:][None, :]
    h = jnp.maximum(h, 0.0)
    h1s_ref[0] = h[:, 0:64]
    h1s_ref[1] = h[:, 64:EMB]
    A1 = jnp.stack([a1s_ref[:], a1d_ref[:]], axis=1)
    s11_ref[:] = h @ (W1_ref[:] @ A1)


def _tc_mid(acc0, We, be, W0, E0, g0, b0, W1, a1s, a1d):
    return pl.pallas_call(
        _mid_body,
        out_shape=[jax.ShapeDtypeStruct((2, N, 64), jnp.float32),
                   jax.ShapeDtypeStruct((N, 2), jnp.float32)],
    )(acc0, We, be, W0, E0, g0, b0, W1, a1s, a1d)


def _fin_body(acc_ref, W1_ref, E1_ref, g_ref, b_ref, ft_ref, cw_ref, nw_ref,
              fv_ref, fsca_ref, Wv_ref, Ws_ref, bs_ref, pW_ref, pb_ref,
              osc_ref, t_ref):
    A = jnp.concatenate(
        [acc_ref[0, :N, 0:64], acc_ref[1, :N, :]], axis=1)
    Mc = jnp.concatenate([
        W1_ref[:],
        E1_ref[:],
        jnp.zeros((12, EMB), jnp.float32),
    ], axis=0)
    num = A @ Mc
    denom = A[:, 132:133]
    h = num / (denom + 1e-16)
    mu = jnp.mean(h, axis=0, keepdims=True)
    var = jnp.mean((h - mu) * (h - mu), axis=0, keepdims=True)
    h2 = (h - mu) / jnp.sqrt(var + 1e-5) * g_ref[:][None, :] + b_ref[:][None, :]
    osc_ref[:] = h2 @ pW_ref[0:EMB, :]

    ft1 = ft_ref[:] @ pW_ref[EMB:2 * EMB, :]
    ft2 = ft_ref[:] @ pW_ref[2 * EMB:3 * EMB, :]
    ii = lax.broadcasted_iota(jnp.int32, (512, 126), 1)
    ohc = (cw_ref[:][:, None] == ii).astype(jnp.float32)
    ohn = (nw_ref[:][:, None] == ii).astype(jnp.float32)
    tcur = ohc @ ft1
    tnxt = ohn @ ft2
    vo0 = fv_ref[0] @ Wv_ref[:]
    vo1 = fv_ref[1] @ Wv_ref[:]
    vo2 = fv_ref[2] @ Wv_ref[:]
    nsq = vo0 * vo0 + vo1 * vo1 + vo2 * vo2
    vnorm = jnp.sqrt(nsq + 1e-8)
    cat = jnp.concatenate([fsca_ref[:], vnorm], axis=1)
    fs = cat @ Ws_ref[:] + bs_ref[:][None, :]
    fs = jnp.maximum(fs, 0.2 * fs)
    tf = fs @ pW_ref[3 * EMB:, :]
    t_ref[:] = tcur + tnxt + tf + pb_ref[:][None, :]


def _tc_fin(acc1, W1, E1, g1, b1, ft, cw, nw, fv3, fsca, Wv, Ws, bs, pW, pb):
    return pl.pallas_call(
        _fin_body,
        out_shape=[jax.ShapeDtypeStruct((N, 1), jnp.float32),
                   jax.ShapeDtypeStruct((512, 1), jnp.float32)],
    )(acc1, W1, E1, g1, b1, ft, cw, nw, fv3, fsca, Wv, Ws, bs, pW, pb)


_NBLK = 2000


def _tail_body(osc_ref, nb_ref, t_ref, out_ref):
    nbv = nb_ref[0, 0, :]
    ii = lax.broadcasted_iota(jnp.int32, (_NBLK, 512), 1)
    oh = (nbv[:, None] == ii).astype(jnp.float32)
    out_ref[:] = osc_ref[:] + oh @ t_ref[:]


def _tc_tail(osc, nb3, t):
    nblocks = N // _NBLK
    return pl.pallas_call(
        _tail_body,
        grid=(nblocks,),
        in_specs=[
            pl.BlockSpec((_NBLK, 1), lambda i: (i, 0)),
            pl.BlockSpec((1, 1, _NBLK), lambda i: (i, 0, 0)),
            pl.BlockSpec((512, 1), lambda i: (0, 0)),
        ],
        out_specs=pl.BlockSpec((_NBLK, 1), lambda i: (i, 0)),
        out_shape=jax.ShapeDtypeStruct((N, 1), jnp.float32),
    )(osc, nb3, t)


def kernel(node_feat_frags, edge_index_frags, edge_features, current_wid,
           next_motif_wid, focal_sca, focal_vec, node_batch_frags,
           node_emb_W, node_emb_b, gat0_W, gat0_edgeW, gat0_asrc, gat0_adst,
           gat1_W, gat1_edgeW, gat1_asrc, gat1_adst, bn0_g, bn0_b, bn1_g,
           bn1_b, frag_table, focal_Wv, focal_Ws, focal_bs, pred_W, pred_b):
    x = node_feat_frags.astype(jnp.float32)
    src = edge_index_frags[0].astype(jnp.int32)
    dst = edge_index_frags[1].astype(jnp.int32)
    ea = edge_features.astype(jnp.float32)

    pad_e = E_PAD - E
    srcp = jnp.concatenate([src, jnp.zeros((pad_e,), jnp.int32)])
    dstp = jnp.concatenate([dst, jnp.zeros((pad_e,), jnp.int32)])
    eap = jnp.concatenate([ea, jnp.zeros((pad_e, 4), jnp.float32)], axis=0)
    ea_cols = eap.T
    ea16 = jnp.concatenate([
        eap,
        jnp.ones((E_PAD, 1), jnp.float32),
        jnp.zeros((E_PAD, 11), jnp.float32),
    ], axis=1)
    xs0 = jnp.stack([
        x[:, 0:32],
        jnp.concatenate([x[:, 32:45], jnp.zeros((N, 19), jnp.float32)],
                        axis=1),
    ], axis=0)
    fv3 = focal_vec.transpose(2, 0, 1)
    nb3 = node_batch_frags.astype(jnp.int32).reshape(N // _NBLK, 1, _NBLK)

    s01, eal01 = _tc_pre(x, ea_cols, node_emb_W, node_emb_b, gat0_W,
                         gat0_asrc, gat0_adst, gat0_edgeW, gat1_edgeW,
                         gat1_asrc)
    s2flat0 = s01.reshape(-1)

    acc0 = _sc_layer0(xs0, s2flat0, srcp, dstp, eal01[0], ea16)

    h1s, s11 = _tc_mid(acc0, node_emb_W, node_emb_b, gat0_W, gat0_edgeW,
                       bn0_g, bn0_b, gat1_W, gat1_asrc, gat1_adst)
    s2flat1 = s11.reshape(-1)

    acc1 = _sc_layer1(h1s, s2flat1, srcp, dstp, eal01[1], ea16)

    osc, t = _tc_fin(acc1, gat1_W, gat1_edgeW, bn1_g, bn1_b, frag_table,
                     current_wid.astype(jnp.int32),
                     next_motif_wid.astype(jnp.int32), fv3, focal_sca,
                     focal_Wv, focal_Ws, focal_bs, pred_W, pred_b)

    return _tc_tail(osc, nb3, t)
